# baseline (speedup 1.0000x reference)
import functools

import jax
import jax.numpy as jnp
from jax import lax
from jax.experimental import pallas as pl
from jax.experimental.pallas import tpu as pltpu
from jax.experimental.pallas import tpu_sc as plsc

N_E = 160000
N_LG = 2560000
ROWS = 4000
GRID = N_E // ROWS

NC, NS = 2, 16
HALF = N_E // NC
ACCR = 80128
BCH = 624
IW = 512
IROWS = N_LG // IW
CHR = 8
NBUF = 3
NCHUNK = IROWS // CHR
GSTEPS = (NCHUNK + NS - 1) // NS


def _edge_proj_body(x_ref, w1_ref, b1_ref, w2a_ref, w2b_ref, b2a_ref,
                    b2b_ref, h0_ref, h1_ref):
    x = x_ref[...]
    t = jnp.maximum(jnp.dot(x, w1_ref[...], preferred_element_type=jnp.float32)
                    + b1_ref[...], 0.0)
    h0_ref[...] = (jnp.dot(t, w2a_ref[...], preferred_element_type=jnp.float32)
                   + b2a_ref[...])
    h1_ref[...] = (jnp.dot(t, w2b_ref[...], preferred_element_type=jnp.float32)
                   + b2b_ref[...])


def _edge_proj(x, w1, b1, w2, b2):
    full = lambda s: pl.BlockSpec(s, lambda i: (0,) * len(s))
    row16 = pl.BlockSpec((ROWS, 16), lambda i: (i, 0))
    return pl.pallas_call(
        _edge_proj_body,
        grid=(GRID,),
        in_specs=[pl.BlockSpec((ROWS, 16), lambda i: (i, 0)),
                  full((16, 128)), full((1, 128)),
                  full((128, 16)), full((128, 16)),
                  full((1, 16)), full((1, 16))],
        out_specs=[row16, row16],
        out_shape=[jax.ShapeDtypeStruct((N_E, 16), jnp.float32),
                   jax.ShapeDtypeStruct((N_E, 16), jnp.float32)],
    )(x, w1, b1.reshape(1, 128), w2[:, :16], w2[:, 16:],
      b2[:16].reshape(1, 16), b2[16:].reshape(1, 16))


def _update_body(h0_ref, h1_ref, a0_ref, a1_ref, cnt_ref,
                 w1a_ref, w1b_ref, w1c_ref, w1d_ref, b1_ref,
                 w2a_ref, w2b_ref, b2a_ref, b2b_ref,
                 g0_ref, g1_ref, bb0_ref, bb1_ref, o0_ref, o1_ref):
    h0 = h0_ref[...]
    h1 = h1_ref[...]
    inv_denom = 1.0 / jnp.maximum(cnt_ref[...], 1.0)
    a0 = a0_ref[...] * inv_denom
    a1 = a1_ref[...] * inv_denom
    dot = lambda a, b: jnp.dot(a, b, preferred_element_type=jnp.float32)
    t = jnp.maximum(dot(h0, w1a_ref[...]) + dot(h1, w1b_ref[...])
                    + dot(a0, w1c_ref[...]) + dot(a1, w1d_ref[...])
                    + b1_ref[...], 0.0)
    r0 = h0 + dot(t, w2a_ref[...]) + b2a_ref[...]
    r1 = h1 + dot(t, w2b_ref[...]) + b2b_ref[...]
    ones16 = jnp.ones((16, 16), jnp.float32)
    m = (dot(r0, ones16) + dot(r1, ones16)) * (1.0 / 32.0)
    q = (dot(r0 * r0, ones16) + dot(r1 * r1, ones16)) * (1.0 / 32.0)
    inv = jax.lax.rsqrt(q - m * m + 1e-6)
    o0_ref[...] = (r0 - m) * inv * g0_ref[...] + bb0_ref[...]
    o1_ref[...] = (r1 - m) * inv * g1_ref[...] + bb1_ref[...]


def _update(h0, h1, a0, a1, cnt, w1, b1, w2, b2, g, b):
    full = lambda s: pl.BlockSpec(s, lambda i: (0,) * len(s))
    row16 = pl.BlockSpec((ROWS, 16), lambda i: (i, 0))
    return pl.pallas_call(
        _update_body,
        grid=(GRID,),
        in_specs=[row16, row16, row16, row16,
                  pl.BlockSpec((ROWS, 1), lambda i: (i, 0))]
                 + [full((16, 128))] * 4 + [full((1, 128))]
                 + [full((128, 16))] * 2 + [full((1, 16))] * 6,
        out_specs=[row16, row16],
        out_shape=[jax.ShapeDtypeStruct((N_E, 16), jnp.float32),
                   jax.ShapeDtypeStruct((N_E, 16), jnp.float32)],
    )(h0, h1, a0, a1, cnt.reshape(N_E, 1),
      w1[:16], w1[16:32], w1[32:48], w1[48:], b1.reshape(1, 128),
      w2[:, :16], w2[:, 16:],
      b2[:16].reshape(1, 16), b2[16:].reshape(1, 16),
      g[:16].reshape(1, 16), g[16:].reshape(1, 16),
      b[:16].reshape(1, 16), b[16:].reshape(1, 16))


def _out_mlp_body(h0_ref, h1_ref, w1a_ref, w1b_ref, b1_ref, w2_ref, b2_ref,
                  o_ref):
    dot = lambda a, b: jnp.dot(a, b, preferred_element_type=jnp.float32)
    t = jnp.maximum(dot(h0_ref[...], w1a_ref[...])
                    + dot(h1_ref[...], w1b_ref[...]) + b1_ref[...], 0.0)
    o_ref[...] = dot(t, w2_ref[...]) + b2_ref[...]


def _out_mlp(h0, h1, w1, b1, w2, b2):
    full = lambda s: pl.BlockSpec(s, lambda i: (0,) * len(s))
    row16 = pl.BlockSpec((ROWS, 16), lambda i: (i, 0))
    return pl.pallas_call(
        _out_mlp_body,
        grid=(GRID,),
        in_specs=[row16, row16, full((16, 128)), full((16, 128)),
                  full((1, 128)), full((128, 1)), full((1, 1))],
        out_specs=pl.BlockSpec((ROWS, 1), lambda i: (i, 0)),
        out_shape=jax.ShapeDtypeStruct((N_E, 1), jnp.float32),
    )(h0, h1, w1[:16], w1[16:], b1.reshape(1, 128), w2,
      b2.reshape(1, 1))


def _dst_local_body(d_ref, o_ref):
    pid = pl.program_id(0)
    base = jnp.where(pid >= IROWS // 1000, HALF, 0).astype(jnp.int32)
    d = d_ref[...]
    inr = (d >= base) & (d < base + HALF)
    o_ref[...] = jnp.where(inr, d - base, HALF + 1 + (d % 127))


def _dst_local(dst2d):
    return pl.pallas_call(
        _dst_local_body,
        grid=(2 * (IROWS // 1000),),
        in_specs=[pl.BlockSpec((1000, IW),
                               lambda r: (r % (IROWS // 1000), 0))],
        out_specs=pl.BlockSpec((1000, IW), lambda r: (r, 0)),
        out_shape=jax.ShapeDtypeStruct((2 * IROWS, IW), jnp.int32),
    )(dst2d)


def _make_seg_sum(with_cnt):
    out_type = [jax.ShapeDtypeStruct((N_E, 16), jnp.float32),
                jax.ShapeDtypeStruct((N_E, 16), jnp.float32)]
    if with_cnt:
        out_type.append(jax.ShapeDtypeStruct((N_E,), jnp.float32))

    nbuf = 3 if with_cnt else 4
    bch = 624 if with_cnt else 312

    scratch = [
        pltpu.VMEM_SHARED((ACCR, 16), jnp.float32),
        pltpu.VMEM((CHR, IW), jnp.int32),
        pltpu.VMEM((CHR, IW), jnp.int32),
        pltpu.VMEM((nbuf, IW, 16), jnp.float32),
        pltpu.VMEM((bch, 16), jnp.float32),
    ]
    if with_cnt:
        scratch += [
            pltpu.VMEM_SHARED((ACCR,), jnp.float32),
            pltpu.VMEM((IW,), jnp.float32),
            pltpu.VMEM((bch,), jnp.float32),
        ]
    scratch += [pltpu.SemaphoreType.DMA] * (3 * nbuf)

    def body(h0, h1, src2d, dloc, *refs):
        nout = 3 if with_cnt else 2
        outs = refs[:nout]
        acc, idxs, idxd, rows, bounce = refs[nout:nout + 5]
        k = nout + 5
        if with_cnt:
            cacc, ones, bounce1 = refs[k:k + 3]
            k += 3
        else:
            cacc = ones = bounce1 = None
        gsem = refs[k:k + nbuf]
        ssem = refs[k + nbuf:k + 2 * nbuf]
        csem = refs[k + 2 * nbuf:]
        c = lax.axis_index("c")
        s = lax.axis_index("s")
        zrow = jnp.zeros((16,), jnp.float32)
        if with_cnt:
            for k in range(IW // 16):
                ones[pl.ds(k * 16, 16)] = jnp.ones((16,), jnp.float32)

        for q in range(2):
            hq = h0 if q == 0 else h1
            outq = outs[q]
            do_cnt = with_cnt and q == 0

            nz = 5008 // bch
            def _zb(i, carry):
                bounce[i, :] = zrow
                return carry
            lax.fori_loop(0, bch, _zb, 0)
            for kk in range(nz):
                pltpu.sync_copy(bounce,
                                acc.at[pl.ds(s * 5008 + kk * bch, bch)])
            pltpu.sync_copy(bounce.at[pl.ds(0, 16)],
                            acc.at[pl.ds(s * 5008 + nz * bch, 16)])
            if do_cnt:
                def _zb1(i, carry):
                    bounce1[pl.ds(i * 16, 16)] = zrow
                    return carry
                lax.fori_loop(0, bch // 16, _zb1, 0)
                for kk in range(nz):
                    pltpu.sync_copy(bounce1,
                                    cacc.at[pl.ds(s * 5008 + kk * bch, bch)])
                pltpu.sync_copy(bounce1.at[pl.ds(0, 16)],
                                cacc.at[pl.ds(s * 5008 + nz * bch, 16)])
            plsc.subcore_barrier()

            def chunk(gs, carry):
                t = gs * NS + s

                @pl.when(t < NCHUNK)
                def _():
                    row0 = t * CHR
                    pltpu.sync_copy(src2d.at[pl.ds(row0, CHR)], idxs)
                    pltpu.sync_copy(dloc.at[pl.ds(c * IROWS + row0, CHR)],
                                    idxd)
                    for b in range(nbuf):
                        pltpu.async_copy(hq.at[idxs.at[b]], rows.at[b],
                                         gsem[b])
                    for j in range(CHR):
                        b = j % nbuf
                        pltpu.make_async_copy(
                            hq.at[idxs.at[j]], rows.at[b], gsem[b]).wait()
                        pltpu.async_copy(rows.at[b], acc.at[idxd.at[j]],
                                         ssem[b], add=True)
                        if do_cnt:
                            pltpu.async_copy(ones, cacc.at[idxd.at[j]],
                                             csem[b], add=True)
                        nj = j + nbuf
                        if nj < CHR:
                            pltpu.make_async_copy(
                                rows.at[b], acc.at[idxd.at[j]],
                                ssem[b]).wait()
                            if do_cnt:
                                pltpu.make_async_copy(
                                    ones, cacc.at[idxd.at[j]],
                                    csem[b]).wait()
                            pltpu.async_copy(hq.at[idxs.at[nj]],
                                             rows.at[b], gsem[b])
                    for j in range(CHR - nbuf, CHR):
                        b = j % nbuf
                        pltpu.make_async_copy(
                            rows.at[b], acc.at[idxd.at[j]], ssem[b]).wait()
                        if do_cnt:
                            pltpu.make_async_copy(
                                ones, cacc.at[idxd.at[j]], csem[b]).wait()
                return carry
            lax.fori_loop(0, GSTEPS, chunk, 0)
            plsc.subcore_barrier()

            nd = 5000 // bch
            for kk in range(nd + 1):
                n = bch if kk < nd else 5000 - nd * bch
                a_off = s * 5000 + kk * bch
                o_off = c * HALF + s * 5000 + kk * bch
                pltpu.sync_copy(acc.at[pl.ds(a_off, n)],
                                bounce.at[pl.ds(0, n)])
                pltpu.sync_copy(bounce.at[pl.ds(0, n)],
                                outq.at[pl.ds(o_off, n)])
                if do_cnt:
                    pltpu.sync_copy(cacc.at[pl.ds(a_off, n)],
                                    bounce1.at[pl.ds(0, n)])
                    pltpu.sync_copy(bounce1.at[pl.ds(0, n)],
                                    outs[2].at[pl.ds(o_off, n)])
            plsc.subcore_barrier()

    mesh = plsc.VectorSubcoreMesh(core_axis_name="c", subcore_axis_name="s")
    return pl.kernel(body, out_type=out_type, mesh=mesh,
                     scratch_types=scratch,
                     compiler_params=pltpu.CompilerParams(
                         use_tc_tiling_on_sc=False))


_seg_sum_cnt = _make_seg_sum(True)
_seg_sum = _make_seg_sum(False)


def kernel(edge_features, lg_src, lg_dst,
           ep_W1, ep_b1, ep_W2, ep_b2,
           up0_W1, up0_b1, up0_W2, up0_b2, ln0_g, ln0_b,
           up1_W1, up1_b1, up1_W2, up1_b2, ln1_g, ln1_b,
           out_W1, out_b1, out_W2, out_b2):
    h0, h1 = _edge_proj(edge_features, ep_W1, ep_b1, ep_W2, ep_b2)

    src2d = lg_src.reshape(IROWS, IW)
    dloc = _dst_local(lg_dst.reshape(IROWS, IW))

    a0, a1, cnt = _seg_sum_cnt(h0, h1, src2d, dloc)
    h0, h1 = _update(h0, h1, a0, a1, cnt,
                     up0_W1, up0_b1, up0_W2, up0_b2, ln0_g, ln0_b)
    a0, a1 = _seg_sum(h0, h1, src2d, dloc)
    h0, h1 = _update(h0, h1, a0, a1, cnt,
                     up1_W1, up1_b1, up1_W2, up1_b2, ln1_g, ln1_b)

    return _out_mlp(h0, h1, out_W1, out_b1, out_W2, out_b2)


# speedup vs baseline: 1.0049x; 1.0049x over previous
import functools

import jax
import jax.numpy as jnp
from jax import lax
from jax.experimental import pallas as pl
from jax.experimental.pallas import tpu as pltpu
from jax.experimental.pallas import tpu_sc as plsc

N_E = 160000
N_LG = 2560000
ROWS = 4000
GRID = N_E // ROWS

NC, NS = 2, 16
HALF = N_E // NC
ACCR = 80128
BCH = 624
IW = 512
IROWS = N_LG // IW
CHR = 8
NBUF = 3
NCHUNK = IROWS // CHR
GSTEPS = (NCHUNK + NS - 1) // NS


def _edge_proj_body(x_ref, w1_ref, b1_ref, w2a_ref, w2b_ref, b2a_ref,
                    b2b_ref, h0_ref, h1_ref):
    x = x_ref[...]
    t = jnp.maximum(jnp.dot(x, w1_ref[...], preferred_element_type=jnp.float32)
                    + b1_ref[...], 0.0)
    h0_ref[...] = (jnp.dot(t, w2a_ref[...], preferred_element_type=jnp.float32)
                   + b2a_ref[...])
    h1_ref[...] = (jnp.dot(t, w2b_ref[...], preferred_element_type=jnp.float32)
                   + b2b_ref[...])


def _edge_proj(x, w1, b1, w2, b2):
    full = lambda s: pl.BlockSpec(s, lambda i: (0,) * len(s))
    row16 = pl.BlockSpec((ROWS, 16), lambda i: (i, 0))
    return pl.pallas_call(
        _edge_proj_body,
        grid=(GRID,),
        in_specs=[pl.BlockSpec((ROWS, 16), lambda i: (i, 0)),
                  full((16, 128)), full((1, 128)),
                  full((128, 16)), full((128, 16)),
                  full((1, 16)), full((1, 16))],
        out_specs=[row16, row16],
        out_shape=[jax.ShapeDtypeStruct((N_E, 16), jnp.float32),
                   jax.ShapeDtypeStruct((N_E, 16), jnp.float32)],
    )(x, w1, b1.reshape(1, 128), w2[:, :16], w2[:, 16:],
      b2[:16].reshape(1, 16), b2[16:].reshape(1, 16))


def _update_body(h0_ref, h1_ref, a0_ref, a1_ref, cnt_ref,
                 w1a_ref, w1b_ref, w1c_ref, w1d_ref, b1_ref,
                 w2a_ref, w2b_ref, b2a_ref, b2b_ref,
                 g0_ref, g1_ref, bb0_ref, bb1_ref, o0_ref, o1_ref):
    h0 = h0_ref[...]
    h1 = h1_ref[...]
    inv_denom = 1.0 / jnp.maximum(cnt_ref[...], 1.0)
    a0 = a0_ref[...] * inv_denom
    a1 = a1_ref[...] * inv_denom
    dot = lambda a, b: jnp.dot(a, b, preferred_element_type=jnp.float32)
    t = jnp.maximum(dot(h0, w1a_ref[...]) + dot(h1, w1b_ref[...])
                    + dot(a0, w1c_ref[...]) + dot(a1, w1d_ref[...])
                    + b1_ref[...], 0.0)
    r0 = h0 + dot(t, w2a_ref[...]) + b2a_ref[...]
    r1 = h1 + dot(t, w2b_ref[...]) + b2b_ref[...]
    m = (jnp.sum(r0, axis=1, keepdims=True)
         + jnp.sum(r1, axis=1, keepdims=True)) * (1.0 / 32.0)
    d0 = r0 - m
    d1 = r1 - m
    v = (jnp.sum(d0 * d0, axis=1, keepdims=True)
         + jnp.sum(d1 * d1, axis=1, keepdims=True)) * (1.0 / 32.0)
    inv = jax.lax.rsqrt(v + 1e-6)
    o0_ref[...] = d0 * inv * g0_ref[...] + bb0_ref[...]
    o1_ref[...] = d1 * inv * g1_ref[...] + bb1_ref[...]


def _update(h0, h1, a0, a1, cnt, w1, b1, w2, b2, g, b):
    full = lambda s: pl.BlockSpec(s, lambda i: (0,) * len(s))
    row16 = pl.BlockSpec((ROWS, 16), lambda i: (i, 0))
    return pl.pallas_call(
        _update_body,
        grid=(GRID,),
        in_specs=[row16, row16, row16, row16,
                  pl.BlockSpec((ROWS, 1), lambda i: (i, 0))]
                 + [full((16, 128))] * 4 + [full((1, 128))]
                 + [full((128, 16))] * 2 + [full((1, 16))] * 6,
        out_specs=[row16, row16],
        out_shape=[jax.ShapeDtypeStruct((N_E, 16), jnp.float32),
                   jax.ShapeDtypeStruct((N_E, 16), jnp.float32)],
    )(h0, h1, a0, a1, cnt.reshape(N_E, 1),
      w1[:16], w1[16:32], w1[32:48], w1[48:], b1.reshape(1, 128),
      w2[:, :16], w2[:, 16:],
      b2[:16].reshape(1, 16), b2[16:].reshape(1, 16),
      g[:16].reshape(1, 16), g[16:].reshape(1, 16),
      b[:16].reshape(1, 16), b[16:].reshape(1, 16))


def _out_mlp_body(h0_ref, h1_ref, w1a_ref, w1b_ref, b1_ref, w2_ref, b2_ref,
                  o_ref):
    dot = lambda a, b: jnp.dot(a, b, preferred_element_type=jnp.float32)
    t = jnp.maximum(dot(h0_ref[...], w1a_ref[...])
                    + dot(h1_ref[...], w1b_ref[...]) + b1_ref[...], 0.0)
    o_ref[...] = dot(t, w2_ref[...]) + b2_ref[...]


def _out_mlp(h0, h1, w1, b1, w2, b2):
    full = lambda s: pl.BlockSpec(s, lambda i: (0,) * len(s))
    row16 = pl.BlockSpec((ROWS, 16), lambda i: (i, 0))
    return pl.pallas_call(
        _out_mlp_body,
        grid=(GRID,),
        in_specs=[row16, row16, full((16, 128)), full((16, 128)),
                  full((1, 128)), full((128, 1)), full((1, 1))],
        out_specs=pl.BlockSpec((ROWS, 1), lambda i: (i, 0)),
        out_shape=jax.ShapeDtypeStruct((N_E, 1), jnp.float32),
    )(h0, h1, w1[:16], w1[16:], b1.reshape(1, 128), w2,
      b2.reshape(1, 1))


def _dst_local_body(d_ref, o_ref):
    pid = pl.program_id(0)
    base = jnp.where(pid >= IROWS // 1000, HALF, 0).astype(jnp.int32)
    d = d_ref[...]
    inr = (d >= base) & (d < base + HALF)
    o_ref[...] = jnp.where(inr, d - base, HALF + 1 + (d % 127))


def _dst_local(dst2d):
    return pl.pallas_call(
        _dst_local_body,
        grid=(2 * (IROWS // 1000),),
        in_specs=[pl.BlockSpec((1000, IW),
                               lambda r: (r % (IROWS // 1000), 0))],
        out_specs=pl.BlockSpec((1000, IW), lambda r: (r, 0)),
        out_shape=jax.ShapeDtypeStruct((2 * IROWS, IW), jnp.int32),
    )(dst2d)


def _make_seg_sum(with_cnt):
    out_type = [jax.ShapeDtypeStruct((N_E, 16), jnp.float32),
                jax.ShapeDtypeStruct((N_E, 16), jnp.float32)]
    if with_cnt:
        out_type.append(jax.ShapeDtypeStruct((N_E,), jnp.float32))

    nbuf = 3 if with_cnt else 4
    bch = 624 if with_cnt else 312

    scratch = [
        pltpu.VMEM_SHARED((ACCR, 16), jnp.float32),
        pltpu.VMEM((CHR, IW), jnp.int32),
        pltpu.VMEM((CHR, IW), jnp.int32),
        pltpu.VMEM((nbuf, IW, 16), jnp.float32),
        pltpu.VMEM((bch, 16), jnp.float32),
    ]
    if with_cnt:
        scratch += [
            pltpu.VMEM_SHARED((ACCR,), jnp.float32),
            pltpu.VMEM((IW,), jnp.float32),
            pltpu.VMEM((bch,), jnp.float32),
        ]
    scratch += [pltpu.SemaphoreType.DMA] * (3 * nbuf)

    def body(h0, h1, src2d, dloc, *refs):
        nout = 3 if with_cnt else 2
        outs = refs[:nout]
        acc, idxs, idxd, rows, bounce = refs[nout:nout + 5]
        k = nout + 5
        if with_cnt:
            cacc, ones, bounce1 = refs[k:k + 3]
            k += 3
        else:
            cacc = ones = bounce1 = None
        gsem = refs[k:k + nbuf]
        ssem = refs[k + nbuf:k + 2 * nbuf]
        csem = refs[k + 2 * nbuf:]
        c = lax.axis_index("c")
        s = lax.axis_index("s")
        zrow = jnp.zeros((16,), jnp.float32)
        if with_cnt:
            for k in range(IW // 16):
                ones[pl.ds(k * 16, 16)] = jnp.ones((16,), jnp.float32)

        for q in range(2):
            hq = h0 if q == 0 else h1
            outq = outs[q]
            do_cnt = with_cnt and q == 0

            nz = 5008 // bch
            def _zb(i, carry):
                bounce[i, :] = zrow
                return carry
            lax.fori_loop(0, bch, _zb, 0)
            for kk in range(nz):
                pltpu.sync_copy(bounce,
                                acc.at[pl.ds(s * 5008 + kk * bch, bch)])
            pltpu.sync_copy(bounce.at[pl.ds(0, 16)],
                            acc.at[pl.ds(s * 5008 + nz * bch, 16)])
            if do_cnt:
                def _zb1(i, carry):
                    bounce1[pl.ds(i * 16, 16)] = zrow
                    return carry
                lax.fori_loop(0, bch // 16, _zb1, 0)
                for kk in range(nz):
                    pltpu.sync_copy(bounce1,
                                    cacc.at[pl.ds(s * 5008 + kk * bch, bch)])
                pltpu.sync_copy(bounce1.at[pl.ds(0, 16)],
                                cacc.at[pl.ds(s * 5008 + nz * bch, 16)])
            plsc.subcore_barrier()

            def chunk(gs, carry):
                t = gs * NS + s

                @pl.when(t < NCHUNK)
                def _():
                    row0 = t * CHR
                    pltpu.sync_copy(src2d.at[pl.ds(row0, CHR)], idxs)
                    pltpu.sync_copy(dloc.at[pl.ds(c * IROWS + row0, CHR)],
                                    idxd)
                    for b in range(nbuf):
                        pltpu.async_copy(hq.at[idxs.at[b]], rows.at[b],
                                         gsem[b])
                    for j in range(CHR):
                        b = j % nbuf
                        pltpu.make_async_copy(
                            hq.at[idxs.at[j]], rows.at[b], gsem[b]).wait()
                        pltpu.async_copy(rows.at[b], acc.at[idxd.at[j]],
                                         ssem[b], add=True)
                        if do_cnt:
                            pltpu.async_copy(ones, cacc.at[idxd.at[j]],
                                             csem[b], add=True)
                        nj = j + nbuf
                        if nj < CHR:
                            pltpu.make_async_copy(
                                rows.at[b], acc.at[idxd.at[j]],
                                ssem[b]).wait()
                            if do_cnt:
                                pltpu.make_async_copy(
                                    ones, cacc.at[idxd.at[j]],
                                    csem[b]).wait()
                            pltpu.async_copy(hq.at[idxs.at[nj]],
                                             rows.at[b], gsem[b])
                    for j in range(CHR - nbuf, CHR):
                        b = j % nbuf
                        pltpu.make_async_copy(
                            rows.at[b], acc.at[idxd.at[j]], ssem[b]).wait()
                        if do_cnt:
                            pltpu.make_async_copy(
                                ones, cacc.at[idxd.at[j]], csem[b]).wait()
                return carry
            lax.fori_loop(0, GSTEPS, chunk, 0)
            plsc.subcore_barrier()

            nd = 5000 // bch
            for kk in range(nd + 1):
                n = bch if kk < nd else 5000 - nd * bch
                a_off = s * 5000 + kk * bch
                o_off = c * HALF + s * 5000 + kk * bch
                pltpu.sync_copy(acc.at[pl.ds(a_off, n)],
                                bounce.at[pl.ds(0, n)])
                pltpu.sync_copy(bounce.at[pl.ds(0, n)],
                                outq.at[pl.ds(o_off, n)])
                if do_cnt:
                    pltpu.sync_copy(cacc.at[pl.ds(a_off, n)],
                                    bounce1.at[pl.ds(0, n)])
                    pltpu.sync_copy(bounce1.at[pl.ds(0, n)],
                                    outs[2].at[pl.ds(o_off, n)])
            plsc.subcore_barrier()

    mesh = plsc.VectorSubcoreMesh(core_axis_name="c", subcore_axis_name="s")
    return pl.kernel(body, out_type=out_type, mesh=mesh,
                     scratch_types=scratch,
                     compiler_params=pltpu.CompilerParams(
                         use_tc_tiling_on_sc=False))


_seg_sum_cnt = _make_seg_sum(True)
_seg_sum = _make_seg_sum(False)


def kernel(edge_features, lg_src, lg_dst,
           ep_W1, ep_b1, ep_W2, ep_b2,
           up0_W1, up0_b1, up0_W2, up0_b2, ln0_g, ln0_b,
           up1_W1, up1_b1, up1_W2, up1_b2, ln1_g, ln1_b,
           out_W1, out_b1, out_W2, out_b2):
    h0, h1 = _edge_proj(edge_features, ep_W1, ep_b1, ep_W2, ep_b2)

    src2d = lg_src.reshape(IROWS, IW)
    dloc = _dst_local(lg_dst.reshape(IROWS, IW))

    a0, a1, cnt = _seg_sum_cnt(h0, h1, src2d, dloc)
    h0, h1 = _update(h0, h1, a0, a1, cnt,
                     up0_W1, up0_b1, up0_W2, up0_b2, ln0_g, ln0_b)
    a0, a1 = _seg_sum(h0, h1, src2d, dloc)
    h0, h1 = _update(h0, h1, a0, a1, cnt,
                     up1_W1, up1_b1, up1_W2, up1_b2, ln1_g, ln1_b)

    return _out_mlp(h0, h1, out_W1, out_b1, out_W2, out_b2)
